# Initial kernel scaffold; baseline (speedup 1.0000x reference)
import jax
import jax.numpy as jnp
from jax.experimental import pallas as pl


def kernel(x, edge_index, batch, W_in, b_in, Ws, bs, gammas, betas, bn_means, bn_vars, W1, b1, W2, b2):
    raise NotImplementedError("write your pallas kernel here")


# speedup vs baseline: 22.0212x; 22.0212x over previous
import functools

import jax
import jax.numpy as jnp
from jax import lax
from jax.experimental import pallas as pl
from jax.experimental.pallas import tpu as pltpu
from jax.experimental.pallas import tpu_sc as plsc

N = 10000
E = 320000
H = 128
G = 64
L = 4

NC = 2
NS = 16
NW = NC * NS
CH = 128
NCHUNK = 79
PH = 40
EPAD = NW * NCHUNK * CH
NPAD = 10112
RPT = NPAD // NS
ZR = 32

_HIGH = jax.lax.Precision.HIGHEST


def _sc_mesh():
    return plsc.VectorSubcoreMesh(core_axis_name="c", subcore_axis_name="s")


def _deg_body(dst_hbm, out_hbm, dstall, ones_v, zv, deg_sh, sem, semz):
    cid = lax.axis_index("c")
    sid = lax.axis_index("s")
    wid = sid * NC + cid

    for k in range(RPT // 16 + 1):
        zv[pl.ds(k * 16, 16)] = jnp.zeros((16,), jnp.float32)
    for k in range(CH // 16):
        ones_v[pl.ds(k * 16, 16)] = jnp.ones((16,), jnp.float32)
    pltpu.async_copy(zv.at[pl.ds(0, RPT)],
                     deg_sh.at[pl.ds(sid * RPT, RPT)], semz).wait()
    plsc.subcore_barrier()

    pltpu.sync_copy(dst_hbm.at[wid], dstall)

    def body(c, _):
        pltpu.async_copy(ones_v, deg_sh.at[dstall.at[c]], sem, add=True)
        return 0
    lax.fori_loop(0, NCHUNK, body, 0)

    def drain(c, _):
        pltpu.make_async_copy(ones_v, deg_sh.at[dstall.at[0]], sem).wait()
        return 0
    lax.fori_loop(0, NCHUNK, drain, 0)
    plsc.subcore_barrier()

    pltpu.sync_copy(deg_sh.at[pl.ds(sid * RPT, RPT)], zv.at[pl.ds(0, RPT)])
    pltpu.sync_copy(zv.at[pl.ds(0, RPT)],
                    out_hbm.at[pl.ds(cid * NPAD + sid * RPT, RPT)])


def _sc_degree(dst3):
    kfn = pl.kernel(
        _deg_body,
        out_type=jax.ShapeDtypeStruct((NC * NPAD,), jnp.float32),
        mesh=_sc_mesh(),
        scratch_types=[
            pltpu.VMEM((NCHUNK, CH), jnp.int32),
            pltpu.VMEM((CH,), jnp.float32),
            pltpu.VMEM(((RPT // 16 + 1) * 16,), jnp.float32),
            pltpu.VMEM_SHARED((NPAD,), jnp.float32),
            pltpu.SemaphoreType.DMA,
            pltpu.SemaphoreType.DMA,
        ],
    )
    return kfn(dst3)


def _agg_body(g_hbm, src_hbm, dst_hbm, out_hbm,
              srcall, dstall, rows0, rows1, s_sh,
              sem_g0, sem_g1, sem_s0, sem_s1, semz):
    cid = lax.axis_index("c")
    sid = lax.axis_index("s")
    wid = sid * NC + cid

    for r in range(ZR):
        for k in range(8):
            rows0[r, pl.ds(k * 16, 16)] = jnp.zeros((16,), jnp.float32)

    def zbody(t, _):
        pltpu.async_copy(rows0.at[pl.ds(0, ZR)],
                         s_sh.at[pl.ds(sid * RPT + t * ZR, ZR)], semz)
        return 0
    nfull = RPT // ZR
    lax.fori_loop(0, nfull, zbody, 0)
    tail = RPT - nfull * ZR
    if tail:
        pltpu.async_copy(rows0.at[pl.ds(0, tail)],
                         s_sh.at[pl.ds(sid * RPT + nfull * ZR, tail)], semz)
    for _ in range(nfull):
        pltpu.make_async_copy(rows0.at[pl.ds(0, ZR)],
                              s_sh.at[pl.ds(0, ZR)], semz).wait()
    if tail:
        pltpu.make_async_copy(rows0.at[pl.ds(0, tail)],
                              s_sh.at[pl.ds(0, tail)], semz).wait()

    plsc.subcore_barrier()

    def gather(c, rows, sem):
        return pltpu.async_copy(g_hbm.at[srcall.at[c]], rows, sem)

    def wait_g(rows, sem):
        pltpu.make_async_copy(g_hbm.at[srcall.at[0]], rows, sem).wait()

    def scat(c, rows, sem):
        return pltpu.async_copy(rows, s_sh.at[dstall.at[c]], sem, add=True)

    def wait_s(rows, sem):
        pltpu.make_async_copy(rows, s_sh.at[dstall.at[0]], sem).wait()

    def run_phase(base, n):
        pltpu.sync_copy(src_hbm.at[wid, pl.ds(base, n)],
                        srcall.at[pl.ds(0, n)])
        pltpu.sync_copy(dst_hbm.at[wid, pl.ds(base, n)],
                        dstall.at[pl.ds(0, n)])
        gather(0, rows0, sem_g0)
        if n > 1:
            gather(1, rows1, sem_g1)
        wait_g(rows0, sem_g0)
        scat(0, rows0, sem_s0)

        def body(j, _):
            c1 = 2 * j + 1
            c2 = 2 * j + 2
            wait_g(rows1, sem_g1)
            scat(c1, rows1, sem_s1)
            wait_s(rows0, sem_s0)
            gather(c2, rows0, sem_g0)
            wait_g(rows0, sem_g0)
            scat(c2, rows0, sem_s0)

            @pl.when(c2 + 1 < n)
            def _():
                wait_s(rows1, sem_s1)
                gather(c2 + 1, rows1, sem_g1)
            return 0

        lax.fori_loop(0, (n - 1) // 2, body, 0)
        if n > 1 and n % 2 == 0:
            wait_g(rows1, sem_g1)
            scat(n - 1, rows1, sem_s1)
        if n > 1:
            wait_s(rows1, sem_s1)
        wait_s(rows0, sem_s0)

    run_phase(0, PH)
    run_phase(PH, NCHUNK - PH)
    plsc.subcore_barrier()

    npc = (RPT + CH - 1) // CH
    for t in range(npc):
        rows = rows0 if t % 2 == 0 else rows1
        sem_r = sem_g0 if t % 2 == 0 else sem_g1
        sem_w = sem_s0 if t % 2 == 0 else sem_s1
        nrows = min(CH, RPT - t * CH)
        if t >= 2:
            pltpu.make_async_copy(
                rows.at[pl.ds(0, min(CH, RPT - (t - 2) * CH))],
                out_hbm.at[cid, pl.ds(0, min(CH, RPT - (t - 2) * CH))],
                sem_w).wait()
        pltpu.async_copy(s_sh.at[pl.ds(sid * RPT + t * CH, nrows)],
                         rows.at[pl.ds(0, nrows)], sem_r).wait()
        pltpu.async_copy(rows.at[pl.ds(0, nrows)],
                         out_hbm.at[cid, pl.ds(sid * RPT + t * CH, nrows)],
                         sem_w)
    for t in (npc - 2, npc - 1):
        rows = rows0 if t % 2 == 0 else rows1
        sem_w = sem_s0 if t % 2 == 0 else sem_s1
        nrows = min(CH, RPT - t * CH)
        pltpu.make_async_copy(rows.at[pl.ds(0, nrows)],
                              out_hbm.at[cid, pl.ds(0, nrows)], sem_w).wait()


def _sc_aggregate(g, src3, dst3):
    kfn = pl.kernel(
        _agg_body,
        out_type=jax.ShapeDtypeStruct((NC, NPAD, H), jnp.float32),
        mesh=_sc_mesh(),
        scratch_types=[
            pltpu.VMEM((PH, CH), jnp.int32),
            pltpu.VMEM((PH, CH), jnp.int32),
            pltpu.VMEM((CH, H), jnp.float32),
            pltpu.VMEM((CH, H), jnp.float32),
            pltpu.VMEM_SHARED((NPAD, H), jnp.float32),
            pltpu.SemaphoreType.DMA,
            pltpu.SemaphoreType.DMA,
            pltpu.SemaphoreType.DMA,
            pltpu.SemaphoreType.DMA,
            pltpu.SemaphoreType.DMA,
        ],
    )
    return kfn(g, src3, dst3)


RB = 1000
NBLK = N // RB


def _pre_body(x_ref, win_ref, bin_ref, w0_ref, degp_ref, g_ref, dinv_ref):
    deg = degp_ref[:, 0:1] + degp_ref[:, 1:2] + 1.0
    dinv = lax.rsqrt(jnp.maximum(deg, 1.0))
    h = jnp.maximum(
        jnp.dot(x_ref[...], win_ref[...], precision=_HIGH) + bin_ref[...], 0.0)
    g_ref[...] = jnp.dot(h, w0_ref[...], precision=_HIGH) * dinv
    dinv_ref[...] = dinv


def _tc_pre(x, w_in, b_in, w0, degp_t):
    return pl.pallas_call(
        _pre_body,
        grid=(NBLK,),
        in_specs=[
            pl.BlockSpec((RB, H), lambda i: (i, 0)),
            pl.BlockSpec((H, H), lambda i: (0, 0)),
            pl.BlockSpec((1, H), lambda i: (0, 0)),
            pl.BlockSpec((H, H), lambda i: (0, 0)),
            pl.BlockSpec((RB, NC), lambda i: (i, 0)),
        ],
        out_specs=[
            pl.BlockSpec((RB, H), lambda i: (i, 0)),
            pl.BlockSpec((RB, 1), lambda i: (i, 0)),
        ],
        out_shape=[
            jax.ShapeDtypeStruct((N, H), jnp.float32),
            jax.ShapeDtypeStruct((N, 1), jnp.float32),
        ],
    )(x, w_in, b_in, w0, degp_t)


def _mid_body(parts_ref, g_ref, dinv_ref, p_ref, w_ref, out_ref):
    b, mean, var, gamma, beta = [p_ref[k:k + 1, :] for k in range(5)]
    scale = lax.rsqrt(var + 1e-5) * gamma
    shift = (b - mean) * scale + beta
    dinv = dinv_ref[...]
    s = parts_ref[0] + parts_ref[1] + g_ref[...]
    h = jnp.maximum(s * dinv * scale + shift, 0.0)
    out_ref[...] = jnp.dot(h, w_ref[...], precision=_HIGH) * dinv


def _tc_mid(parts, g, dinv, p, w_next):
    return pl.pallas_call(
        _mid_body,
        grid=(NBLK,),
        in_specs=[
            pl.BlockSpec((NC, RB, H), lambda i: (0, i, 0)),
            pl.BlockSpec((RB, H), lambda i: (i, 0)),
            pl.BlockSpec((RB, 1), lambda i: (i, 0)),
            pl.BlockSpec((5, H), lambda i: (0, 0)),
            pl.BlockSpec((H, H), lambda i: (0, 0)),
        ],
        out_specs=pl.BlockSpec((RB, H), lambda i: (i, 0)),
        out_shape=jax.ShapeDtypeStruct((N, H), jnp.float32),
    )(parts, g, dinv, p, w_next)


def _post_body(parts_ref, g_ref, dinv_ref, p_ref, batch_ref,
               w1_ref, b1_ref, w2_ref, b2_ref, out_ref, sums, cnts):
    i = pl.program_id(0)
    b, mean, var, gamma, beta = [p_ref[k:k + 1, :] for k in range(5)]
    scale = lax.rsqrt(var + 1e-5) * gamma
    shift = (b - mean) * scale + beta
    dinv = dinv_ref[...]
    s = parts_ref[0] + parts_ref[1] + g_ref[...]
    h = jnp.maximum(s * dinv * scale + shift, 0.0)

    gid = lax.broadcasted_iota(jnp.int32, (1, G), 1)
    oh = (batch_ref[...] == gid).astype(jnp.float32)
    dn = (((0,), (0,)), ((), ()))
    blk_sums = lax.dot_general(oh, h, dn, precision=_HIGH)
    blk_cnts = lax.dot_general(oh, jnp.ones_like(h), dn, precision=_HIGH)

    @pl.when(i == 0)
    def _():
        sums[...] = jnp.zeros_like(sums)
        cnts[...] = jnp.zeros_like(cnts)

    sums[...] += blk_sums
    cnts[...] += blk_cnts

    @pl.when(i == NBLK - 1)
    def _():
        gemb = sums[...] / jnp.maximum(cnts[...], 1.0)
        o = jnp.maximum(
            jnp.dot(gemb, w1_ref[...], precision=_HIGH) + b1_ref[...], 0.0)
        out_ref[...] = jnp.dot(o, w2_ref[...], precision=_HIGH) + b2_ref[...]


def _tc_post(parts, g, dinv, p, batch2, w1, b1, w2, b2, n_cls):
    return pl.pallas_call(
        _post_body,
        grid=(NBLK,),
        in_specs=[
            pl.BlockSpec((NC, RB, H), lambda i: (0, i, 0)),
            pl.BlockSpec((RB, H), lambda i: (i, 0)),
            pl.BlockSpec((RB, 1), lambda i: (i, 0)),
            pl.BlockSpec((5, H), lambda i: (0, 0)),
            pl.BlockSpec((RB, 1), lambda i: (i, 0)),
            pl.BlockSpec((H, H // 2), lambda i: (0, 0)),
            pl.BlockSpec((1, H // 2), lambda i: (0, 0)),
            pl.BlockSpec((H // 2, n_cls), lambda i: (0, 0)),
            pl.BlockSpec((1, n_cls), lambda i: (0, 0)),
        ],
        out_specs=pl.BlockSpec((G, n_cls), lambda i: (0, 0)),
        out_shape=jax.ShapeDtypeStruct((G, n_cls), jnp.float32),
        scratch_shapes=[
            pltpu.VMEM((G, H), jnp.float32),
            pltpu.VMEM((G, H), jnp.float32),
        ],
        compiler_params=pltpu.CompilerParams(
            dimension_semantics=("arbitrary",)),
    )(parts, g, dinv, p, batch2, w1, b1, w2, b2)


def kernel(x, edge_index, batch, W_in, b_in, Ws, bs, gammas, betas,
           bn_means, bn_vars, W1, b1, W2, b2):
    n_cls = W2.shape[1]
    src = edge_index[0]
    dst = edge_index[1]

    npad_e = EPAD - E
    pad_i = jnp.arange(npad_e, dtype=jnp.int32)
    src_p = jnp.concatenate([src, pad_i % N])
    dst_p = jnp.concatenate([dst, N + (pad_i % (NPAD - N))])
    src3 = src_p.reshape(NW, NCHUNK, CH)
    dst3 = dst_p.reshape(NW, NCHUNK, CH)

    degp = _sc_degree(dst3).reshape(NC, NPAD)
    degp_t = degp.T

    g, dinv = _tc_pre(x, W_in, b_in.reshape(1, H), Ws[0], degp_t)

    ps = [jnp.stack([bs[i], bn_means[i], bn_vars[i], gammas[i], betas[i]])
          for i in range(L)]

    for i in range(L - 1):
        parts = _sc_aggregate(g, src3, dst3)
        g = _tc_mid(parts, g, dinv, ps[i], Ws[i + 1])

    parts = _sc_aggregate(g, src3, dst3)
    out = _tc_post(parts, g, dinv, ps[L - 1], batch.reshape(N, 1),
                   W1, b1.reshape(1, H // 2), W2, b2.reshape(1, n_cls), n_cls)
    return (out, jnp.float32(0.0))


# speedup vs baseline: 26.2212x; 1.1907x over previous
import functools

import jax
import jax.numpy as jnp
from jax import lax
from jax.experimental import pallas as pl
from jax.experimental.pallas import tpu as pltpu
from jax.experimental.pallas import tpu_sc as plsc

N = 10000
E = 320000
H = 128
G = 64
L = 4

NC = 2
NS = 16
NW = NC * NS
CH = 96
NCHUNK = 105
PH = 48
PHASES = ((0, 48), (48, 48), (96, 9))
EPAD = NW * NCHUNK * CH
NPAD = 10112
RPT = NPAD // NS
ZR = 32

_HIGH = jax.lax.Precision.HIGHEST


def _sc_mesh():
    return plsc.VectorSubcoreMesh(core_axis_name="c", subcore_axis_name="s")


def _deg_body(dst_hbm, out_hbm, dstall, ones_v, zv, deg_sh, sem, semz):
    cid = lax.axis_index("c")
    sid = lax.axis_index("s")
    wid = sid * NC + cid

    for k in range(RPT // 16 + 1):
        zv[pl.ds(k * 16, 16)] = jnp.zeros((16,), jnp.float32)
    for k in range(CH // 16):
        ones_v[pl.ds(k * 16, 16)] = jnp.ones((16,), jnp.float32)
    pltpu.async_copy(zv.at[pl.ds(0, RPT)],
                     deg_sh.at[pl.ds(sid * RPT, RPT)], semz).wait()
    plsc.subcore_barrier()

    pltpu.sync_copy(dst_hbm.at[wid], dstall)

    def body(c, _):
        pltpu.async_copy(ones_v, deg_sh.at[dstall.at[c]], sem, add=True)
        return 0
    lax.fori_loop(0, NCHUNK, body, 0)

    def drain(c, _):
        pltpu.make_async_copy(ones_v, deg_sh.at[dstall.at[0]], sem).wait()
        return 0
    lax.fori_loop(0, NCHUNK, drain, 0)
    plsc.subcore_barrier()

    pltpu.sync_copy(deg_sh.at[pl.ds(sid * RPT, RPT)], zv.at[pl.ds(0, RPT)])
    pltpu.sync_copy(zv.at[pl.ds(0, RPT)],
                    out_hbm.at[pl.ds(cid * NPAD + sid * RPT, RPT)])


def _sc_degree(dst3):
    kfn = pl.kernel(
        _deg_body,
        out_type=jax.ShapeDtypeStruct((NC * NPAD,), jnp.float32),
        mesh=_sc_mesh(),
        scratch_types=[
            pltpu.VMEM((NCHUNK, CH), jnp.int32),
            pltpu.VMEM((CH,), jnp.float32),
            pltpu.VMEM(((RPT // 16 + 1) * 16,), jnp.float32),
            pltpu.VMEM_SHARED((NPAD,), jnp.float32),
            pltpu.SemaphoreType.DMA,
            pltpu.SemaphoreType.DMA,
        ],
    )
    return kfn(dst3)


def _agg_body(g_hbm, src_hbm, dst_hbm, out_hbm,
              srcall, dstall, rows0, rows1, rows2, s_sh,
              sem_g0, sem_g1, sem_g2, sem_s0, sem_s1, sem_s2, semz):
    cid = lax.axis_index("c")
    sid = lax.axis_index("s")
    wid = sid * NC + cid

    for r in range(ZR):
        for k in range(8):
            rows0[r, pl.ds(k * 16, 16)] = jnp.zeros((16,), jnp.float32)

    def zbody(t, _):
        pltpu.async_copy(rows0.at[pl.ds(0, ZR)],
                         s_sh.at[pl.ds(sid * RPT + t * ZR, ZR)], semz)
        return 0
    nfull = RPT // ZR
    lax.fori_loop(0, nfull, zbody, 0)
    tail = RPT - nfull * ZR
    if tail:
        pltpu.async_copy(rows0.at[pl.ds(0, tail)],
                         s_sh.at[pl.ds(sid * RPT + nfull * ZR, tail)], semz)
    for _ in range(nfull):
        pltpu.make_async_copy(rows0.at[pl.ds(0, ZR)],
                              s_sh.at[pl.ds(0, ZR)], semz).wait()
    if tail:
        pltpu.make_async_copy(rows0.at[pl.ds(0, tail)],
                              s_sh.at[pl.ds(0, tail)], semz).wait()

    plsc.subcore_barrier()

    R = (rows0, rows1, rows2)
    SG = (sem_g0, sem_g1, sem_g2)
    SS = (sem_s0, sem_s1, sem_s2)

    def gather(c, p):
        pltpu.async_copy(g_hbm.at[srcall.at[c]], R[p], SG[p])

    def wait_g(p):
        pltpu.make_async_copy(g_hbm.at[srcall.at[0]], R[p], SG[p]).wait()

    def scat(c, p):
        pltpu.async_copy(R[p], s_sh.at[dstall.at[c]], SS[p], add=True)

    def wait_s(p):
        pltpu.make_async_copy(R[p], s_sh.at[dstall.at[0]], SS[p]).wait()

    def run_phase(base, n):
        pltpu.sync_copy(src_hbm.at[wid, pl.ds(base, n)],
                        srcall.at[pl.ds(0, n)])
        pltpu.sync_copy(dst_hbm.at[wid, pl.ds(base, n)],
                        dstall.at[pl.ds(0, n)])
        gather(0, 0)
        gather(1, 1)

        def body(j, _):
            a = 3 * j
            wait_g(0)
            scat(a, 0)

            @pl.when(j > 0)
            def _():
                wait_s(2)
            gather(a + 2, 2)
            wait_g(1)
            scat(a + 1, 1)

            @pl.when(a + 3 < n)
            def _():
                wait_s(0)
                gather(a + 3, 0)
            wait_g(2)
            scat(a + 2, 2)

            @pl.when(a + 4 < n)
            def _():
                wait_s(1)
                gather(a + 4, 1)
            return 0

        lax.fori_loop(0, n // 3, body, 0)
        wait_s(0)
        wait_s(1)
        wait_s(2)

    for base, n in PHASES:
        run_phase(base, n)
    plsc.subcore_barrier()

    npc = (RPT + CH - 1) // CH
    for t in range(npc):
        rows = rows0 if t % 2 == 0 else rows1
        sem_r = sem_g0 if t % 2 == 0 else sem_g1
        sem_w = sem_s0 if t % 2 == 0 else sem_s1
        nrows = min(CH, RPT - t * CH)
        if t >= 2:
            pltpu.make_async_copy(
                rows.at[pl.ds(0, min(CH, RPT - (t - 2) * CH))],
                out_hbm.at[cid, pl.ds(0, min(CH, RPT - (t - 2) * CH))],
                sem_w).wait()
        pltpu.async_copy(s_sh.at[pl.ds(sid * RPT + t * CH, nrows)],
                         rows.at[pl.ds(0, nrows)], sem_r).wait()
        pltpu.async_copy(rows.at[pl.ds(0, nrows)],
                         out_hbm.at[cid, pl.ds(sid * RPT + t * CH, nrows)],
                         sem_w)
    for t in (npc - 2, npc - 1):
        rows = rows0 if t % 2 == 0 else rows1
        sem_w = sem_s0 if t % 2 == 0 else sem_s1
        nrows = min(CH, RPT - t * CH)
        pltpu.make_async_copy(rows.at[pl.ds(0, nrows)],
                              out_hbm.at[cid, pl.ds(0, nrows)], sem_w).wait()


def _sc_aggregate(g, src3, dst3):
    kfn = pl.kernel(
        _agg_body,
        out_type=jax.ShapeDtypeStruct((NC, NPAD, H), jnp.float32),
        mesh=_sc_mesh(),
        scratch_types=[
            pltpu.VMEM((PH, CH), jnp.int32),
            pltpu.VMEM((PH, CH), jnp.int32),
            pltpu.VMEM((CH, H), jnp.float32),
            pltpu.VMEM((CH, H), jnp.float32),
            pltpu.VMEM((CH, H), jnp.float32),
            pltpu.VMEM_SHARED((NPAD, H), jnp.float32),
            pltpu.SemaphoreType.DMA,
            pltpu.SemaphoreType.DMA,
            pltpu.SemaphoreType.DMA,
            pltpu.SemaphoreType.DMA,
            pltpu.SemaphoreType.DMA,
            pltpu.SemaphoreType.DMA,
            pltpu.SemaphoreType.DMA,
        ],
    )
    return kfn(g, src3, dst3)


RB = 1000
NBLK = N // RB


def _pre_body(x_ref, win_ref, bin_ref, w0_ref, degp_ref, g_ref, dinv_ref):
    deg = degp_ref[:, 0:1] + degp_ref[:, 1:2] + 1.0
    dinv = lax.rsqrt(jnp.maximum(deg, 1.0))
    h = jnp.maximum(
        jnp.dot(x_ref[...], win_ref[...], precision=_HIGH) + bin_ref[...], 0.0)
    g_ref[...] = jnp.dot(h, w0_ref[...], precision=_HIGH) * dinv
    dinv_ref[...] = dinv


def _tc_pre(x, w_in, b_in, w0, degp_t):
    return pl.pallas_call(
        _pre_body,
        grid=(NBLK,),
        in_specs=[
            pl.BlockSpec((RB, H), lambda i: (i, 0)),
            pl.BlockSpec((H, H), lambda i: (0, 0)),
            pl.BlockSpec((1, H), lambda i: (0, 0)),
            pl.BlockSpec((H, H), lambda i: (0, 0)),
            pl.BlockSpec((RB, NC), lambda i: (i, 0)),
        ],
        out_specs=[
            pl.BlockSpec((RB, H), lambda i: (i, 0)),
            pl.BlockSpec((RB, 1), lambda i: (i, 0)),
        ],
        out_shape=[
            jax.ShapeDtypeStruct((N, H), jnp.float32),
            jax.ShapeDtypeStruct((N, 1), jnp.float32),
        ],
    )(x, w_in, b_in, w0, degp_t)


def _mid_body(parts_ref, g_ref, dinv_ref, p_ref, w_ref, out_ref):
    b, mean, var, gamma, beta = [p_ref[k:k + 1, :] for k in range(5)]
    scale = lax.rsqrt(var + 1e-5) * gamma
    shift = (b - mean) * scale + beta
    dinv = dinv_ref[...]
    s = parts_ref[0] + parts_ref[1] + g_ref[...]
    h = jnp.maximum(s * dinv * scale + shift, 0.0)
    out_ref[...] = jnp.dot(h, w_ref[...], precision=_HIGH) * dinv


def _tc_mid(parts, g, dinv, p, w_next):
    return pl.pallas_call(
        _mid_body,
        grid=(NBLK,),
        in_specs=[
            pl.BlockSpec((NC, RB, H), lambda i: (0, i, 0)),
            pl.BlockSpec((RB, H), lambda i: (i, 0)),
            pl.BlockSpec((RB, 1), lambda i: (i, 0)),
            pl.BlockSpec((5, H), lambda i: (0, 0)),
            pl.BlockSpec((H, H), lambda i: (0, 0)),
        ],
        out_specs=pl.BlockSpec((RB, H), lambda i: (i, 0)),
        out_shape=jax.ShapeDtypeStruct((N, H), jnp.float32),
    )(parts, g, dinv, p, w_next)


def _post_body(parts_ref, g_ref, dinv_ref, p_ref, batch_ref,
               w1_ref, b1_ref, w2_ref, b2_ref, out_ref, sums, cnts):
    i = pl.program_id(0)
    b, mean, var, gamma, beta = [p_ref[k:k + 1, :] for k in range(5)]
    scale = lax.rsqrt(var + 1e-5) * gamma
    shift = (b - mean) * scale + beta
    dinv = dinv_ref[...]
    s = parts_ref[0] + parts_ref[1] + g_ref[...]
    h = jnp.maximum(s * dinv * scale + shift, 0.0)

    gid = lax.broadcasted_iota(jnp.int32, (1, G), 1)
    oh = (batch_ref[...] == gid).astype(jnp.float32)
    dn = (((0,), (0,)), ((), ()))
    blk_sums = lax.dot_general(oh, h, dn, precision=_HIGH)
    blk_cnts = lax.dot_general(oh, jnp.ones_like(h), dn, precision=_HIGH)

    @pl.when(i == 0)
    def _():
        sums[...] = jnp.zeros_like(sums)
        cnts[...] = jnp.zeros_like(cnts)

    sums[...] += blk_sums
    cnts[...] += blk_cnts

    @pl.when(i == NBLK - 1)
    def _():
        gemb = sums[...] / jnp.maximum(cnts[...], 1.0)
        o = jnp.maximum(
            jnp.dot(gemb, w1_ref[...], precision=_HIGH) + b1_ref[...], 0.0)
        out_ref[...] = jnp.dot(o, w2_ref[...], precision=_HIGH) + b2_ref[...]


def _tc_post(parts, g, dinv, p, batch2, w1, b1, w2, b2, n_cls):
    return pl.pallas_call(
        _post_body,
        grid=(NBLK,),
        in_specs=[
            pl.BlockSpec((NC, RB, H), lambda i: (0, i, 0)),
            pl.BlockSpec((RB, H), lambda i: (i, 0)),
            pl.BlockSpec((RB, 1), lambda i: (i, 0)),
            pl.BlockSpec((5, H), lambda i: (0, 0)),
            pl.BlockSpec((RB, 1), lambda i: (i, 0)),
            pl.BlockSpec((H, H // 2), lambda i: (0, 0)),
            pl.BlockSpec((1, H // 2), lambda i: (0, 0)),
            pl.BlockSpec((H // 2, n_cls), lambda i: (0, 0)),
            pl.BlockSpec((1, n_cls), lambda i: (0, 0)),
        ],
        out_specs=pl.BlockSpec((G, n_cls), lambda i: (0, 0)),
        out_shape=jax.ShapeDtypeStruct((G, n_cls), jnp.float32),
        scratch_shapes=[
            pltpu.VMEM((G, H), jnp.float32),
            pltpu.VMEM((G, H), jnp.float32),
        ],
        compiler_params=pltpu.CompilerParams(
            dimension_semantics=("arbitrary",)),
    )(parts, g, dinv, p, batch2, w1, b1, w2, b2)


def kernel(x, edge_index, batch, W_in, b_in, Ws, bs, gammas, betas,
           bn_means, bn_vars, W1, b1, W2, b2):
    n_cls = W2.shape[1]
    src = edge_index[0]
    dst = edge_index[1]

    npad_e = EPAD - E
    pad_i = jnp.arange(npad_e, dtype=jnp.int32)
    src_p = jnp.concatenate([src, pad_i % N])
    dst_p = jnp.concatenate([dst, N + (pad_i % (NPAD - N))])
    src3 = src_p.reshape(NW, NCHUNK, CH)
    dst3 = dst_p.reshape(NW, NCHUNK, CH)

    degp = _sc_degree(dst3).reshape(NC, NPAD)
    degp_t = degp.T

    g, dinv = _tc_pre(x, W_in, b_in.reshape(1, H), Ws[0], degp_t)

    ps = [jnp.stack([bs[i], bn_means[i], bn_vars[i], gammas[i], betas[i]])
          for i in range(L)]

    for i in range(L - 1):
        parts = _sc_aggregate(g, src3, dst3)
        g = _tc_mid(parts, g, dinv, ps[i], Ws[i + 1])

    parts = _sc_aggregate(g, src3, dst3)
    out = _tc_post(parts, g, dinv, ps[L - 1], batch.reshape(N, 1),
                   W1, b1.reshape(1, H // 2), W2, b2.reshape(1, n_cls), n_cls)
    return (out, jnp.float32(0.0))


# speedup vs baseline: 28.6898x; 1.0941x over previous
import functools

import jax
import jax.numpy as jnp
from jax import lax
from jax.experimental import pallas as pl
from jax.experimental.pallas import tpu as pltpu
from jax.experimental.pallas import tpu_sc as plsc

N = 10000
E = 320000
H = 128
G = 64
L = 4

NC = 2
NS = 16
NW = NC * NS
CH = 96
NCHUNK = 105
PH = 48
PHASES = ((0, 48), (48, 48), (96, 9))
EPAD = NW * NCHUNK * CH
NPAD = 10112
RPT = NPAD // NS
ZR = 32

_HIGH = jax.lax.Precision.DEFAULT


def _sc_mesh():
    return plsc.VectorSubcoreMesh(core_axis_name="c", subcore_axis_name="s")


def _deg_body(dst_hbm, out_hbm, dstall, ones_v, zv, deg_sh, sem, semz):
    cid = lax.axis_index("c")
    sid = lax.axis_index("s")
    wid = sid * NC + cid

    for k in range(RPT // 16 + 1):
        zv[pl.ds(k * 16, 16)] = jnp.zeros((16,), jnp.float32)
    for k in range(CH // 16):
        ones_v[pl.ds(k * 16, 16)] = jnp.ones((16,), jnp.float32)
    pltpu.async_copy(zv.at[pl.ds(0, RPT)],
                     deg_sh.at[pl.ds(sid * RPT, RPT)], semz).wait()
    plsc.subcore_barrier()

    pltpu.sync_copy(dst_hbm.at[wid], dstall)

    def body(c, _):
        pltpu.async_copy(ones_v, deg_sh.at[dstall.at[c]], sem, add=True)
        return 0
    lax.fori_loop(0, NCHUNK, body, 0)

    def drain(c, _):
        pltpu.make_async_copy(ones_v, deg_sh.at[dstall.at[0]], sem).wait()
        return 0
    lax.fori_loop(0, NCHUNK, drain, 0)
    plsc.subcore_barrier()

    pltpu.sync_copy(deg_sh.at[pl.ds(sid * RPT, RPT)], zv.at[pl.ds(0, RPT)])
    pltpu.sync_copy(zv.at[pl.ds(0, RPT)],
                    out_hbm.at[pl.ds(cid * NPAD + sid * RPT, RPT)])


def _sc_degree(dst3):
    kfn = pl.kernel(
        _deg_body,
        out_type=jax.ShapeDtypeStruct((NC * NPAD,), jnp.float32),
        mesh=_sc_mesh(),
        scratch_types=[
            pltpu.VMEM((NCHUNK, CH), jnp.int32),
            pltpu.VMEM((CH,), jnp.float32),
            pltpu.VMEM(((RPT // 16 + 1) * 16,), jnp.float32),
            pltpu.VMEM_SHARED((NPAD,), jnp.float32),
            pltpu.SemaphoreType.DMA,
            pltpu.SemaphoreType.DMA,
        ],
    )
    return kfn(dst3)


def _agg_body(g_hbm, src_hbm, dst_hbm, out_hbm,
              srcall, dstall, rows0, rows1, rows2, s_sh,
              sem_g0, sem_g1, sem_g2, sem_s0, sem_s1, sem_s2, semz):
    cid = lax.axis_index("c")
    sid = lax.axis_index("s")
    wid = sid * NC + cid

    for r in range(ZR):
        for k in range(8):
            rows0[r, pl.ds(k * 16, 16)] = jnp.zeros((16,), jnp.float32)

    def zbody(t, _):
        pltpu.async_copy(rows0.at[pl.ds(0, ZR)],
                         s_sh.at[pl.ds(sid * RPT + t * ZR, ZR)], semz)
        return 0
    nfull = RPT // ZR
    lax.fori_loop(0, nfull, zbody, 0)
    tail = RPT - nfull * ZR
    if tail:
        pltpu.async_copy(rows0.at[pl.ds(0, tail)],
                         s_sh.at[pl.ds(sid * RPT + nfull * ZR, tail)], semz)
    for _ in range(nfull):
        pltpu.make_async_copy(rows0.at[pl.ds(0, ZR)],
                              s_sh.at[pl.ds(0, ZR)], semz).wait()
    if tail:
        pltpu.make_async_copy(rows0.at[pl.ds(0, tail)],
                              s_sh.at[pl.ds(0, tail)], semz).wait()

    plsc.subcore_barrier()

    R = (rows0, rows1, rows2)
    SG = (sem_g0, sem_g1, sem_g2)
    SS = (sem_s0, sem_s1, sem_s2)

    def gather(c, p):
        pltpu.async_copy(g_hbm.at[srcall.at[c]], R[p], SG[p])

    def wait_g(p):
        pltpu.make_async_copy(g_hbm.at[srcall.at[0]], R[p], SG[p]).wait()

    def scat(c, p):
        pltpu.async_copy(R[p], s_sh.at[dstall.at[c]], SS[p], add=True)

    def wait_s(p):
        pltpu.make_async_copy(R[p], s_sh.at[dstall.at[0]], SS[p]).wait()

    def run_phase(base, n):
        pltpu.sync_copy(src_hbm.at[wid, pl.ds(base, n)],
                        srcall.at[pl.ds(0, n)])
        pltpu.sync_copy(dst_hbm.at[wid, pl.ds(base, n)],
                        dstall.at[pl.ds(0, n)])
        gather(0, 0)
        gather(1, 1)

        def body(j, _):
            a = 3 * j
            wait_g(0)
            scat(a, 0)

            @pl.when(j > 0)
            def _():
                wait_s(2)
            gather(a + 2, 2)
            wait_g(1)
            scat(a + 1, 1)

            @pl.when(a + 3 < n)
            def _():
                wait_s(0)
                gather(a + 3, 0)
            wait_g(2)
            scat(a + 2, 2)

            @pl.when(a + 4 < n)
            def _():
                wait_s(1)
                gather(a + 4, 1)
            return 0

        lax.fori_loop(0, n // 3, body, 0)
        wait_s(0)
        wait_s(1)
        wait_s(2)

    for base, n in PHASES:
        run_phase(base, n)
    plsc.subcore_barrier()

    npc = (RPT + CH - 1) // CH
    for t in range(npc):
        rows = rows0 if t % 2 == 0 else rows1
        sem_r = sem_g0 if t % 2 == 0 else sem_g1
        sem_w = sem_s0 if t % 2 == 0 else sem_s1
        nrows = min(CH, RPT - t * CH)
        if t >= 2:
            pltpu.make_async_copy(
                rows.at[pl.ds(0, min(CH, RPT - (t - 2) * CH))],
                out_hbm.at[cid, pl.ds(0, min(CH, RPT - (t - 2) * CH))],
                sem_w).wait()
        pltpu.async_copy(s_sh.at[pl.ds(sid * RPT + t * CH, nrows)],
                         rows.at[pl.ds(0, nrows)], sem_r).wait()
        pltpu.async_copy(rows.at[pl.ds(0, nrows)],
                         out_hbm.at[cid, pl.ds(sid * RPT + t * CH, nrows)],
                         sem_w)
    for t in (npc - 2, npc - 1):
        rows = rows0 if t % 2 == 0 else rows1
        sem_w = sem_s0 if t % 2 == 0 else sem_s1
        nrows = min(CH, RPT - t * CH)
        pltpu.make_async_copy(rows.at[pl.ds(0, nrows)],
                              out_hbm.at[cid, pl.ds(0, nrows)], sem_w).wait()


def _sc_aggregate(g, src3, dst3):
    kfn = pl.kernel(
        _agg_body,
        out_type=jax.ShapeDtypeStruct((NC, NPAD, H), jnp.float32),
        mesh=_sc_mesh(),
        scratch_types=[
            pltpu.VMEM((PH, CH), jnp.int32),
            pltpu.VMEM((PH, CH), jnp.int32),
            pltpu.VMEM((CH, H), jnp.float32),
            pltpu.VMEM((CH, H), jnp.float32),
            pltpu.VMEM((CH, H), jnp.float32),
            pltpu.VMEM_SHARED((NPAD, H), jnp.float32),
            pltpu.SemaphoreType.DMA,
            pltpu.SemaphoreType.DMA,
            pltpu.SemaphoreType.DMA,
            pltpu.SemaphoreType.DMA,
            pltpu.SemaphoreType.DMA,
            pltpu.SemaphoreType.DMA,
            pltpu.SemaphoreType.DMA,
        ],
    )
    return kfn(g, src3, dst3)


RB = 2000
NBLK = N // RB


def _pre_body(x_ref, win_ref, bin_ref, w0_ref, degp_ref, g_ref, dinv_ref):
    deg = degp_ref[:, 0:1] + degp_ref[:, 1:2] + 1.0
    dinv = lax.rsqrt(jnp.maximum(deg, 1.0))
    h = jnp.maximum(
        jnp.dot(x_ref[...], win_ref[...], precision=_HIGH) + bin_ref[...], 0.0)
    g_ref[...] = jnp.dot(h, w0_ref[...], precision=_HIGH) * dinv
    dinv_ref[...] = dinv


def _tc_pre(x, w_in, b_in, w0, degp_t):
    return pl.pallas_call(
        _pre_body,
        grid=(NBLK,),
        in_specs=[
            pl.BlockSpec((RB, H), lambda i: (i, 0)),
            pl.BlockSpec((H, H), lambda i: (0, 0)),
            pl.BlockSpec((1, H), lambda i: (0, 0)),
            pl.BlockSpec((H, H), lambda i: (0, 0)),
            pl.BlockSpec((RB, NC), lambda i: (i, 0)),
        ],
        out_specs=[
            pl.BlockSpec((RB, H), lambda i: (i, 0)),
            pl.BlockSpec((RB, 1), lambda i: (i, 0)),
        ],
        out_shape=[
            jax.ShapeDtypeStruct((N, H), jnp.float32),
            jax.ShapeDtypeStruct((N, 1), jnp.float32),
        ],
    )(x, w_in, b_in, w0, degp_t)


def _mid_body(parts_ref, g_ref, dinv_ref, p_ref, w_ref, out_ref):
    b, mean, var, gamma, beta = [p_ref[k:k + 1, :] for k in range(5)]
    scale = lax.rsqrt(var + 1e-5) * gamma
    shift = (b - mean) * scale + beta
    dinv = dinv_ref[...]
    s = parts_ref[0] + parts_ref[1] + g_ref[...]
    h = jnp.maximum(s * dinv * scale + shift, 0.0)
    out_ref[...] = jnp.dot(h, w_ref[...], precision=_HIGH) * dinv


def _tc_mid(parts, g, dinv, p, w_next):
    return pl.pallas_call(
        _mid_body,
        grid=(NBLK,),
        in_specs=[
            pl.BlockSpec((NC, RB, H), lambda i: (0, i, 0)),
            pl.BlockSpec((RB, H), lambda i: (i, 0)),
            pl.BlockSpec((RB, 1), lambda i: (i, 0)),
            pl.BlockSpec((5, H), lambda i: (0, 0)),
            pl.BlockSpec((H, H), lambda i: (0, 0)),
        ],
        out_specs=pl.BlockSpec((RB, H), lambda i: (i, 0)),
        out_shape=jax.ShapeDtypeStruct((N, H), jnp.float32),
    )(parts, g, dinv, p, w_next)


def _post_body(parts_ref, g_ref, dinv_ref, p_ref, batch_ref,
               w1_ref, b1_ref, w2_ref, b2_ref, out_ref, sums, cnts):
    i = pl.program_id(0)
    b, mean, var, gamma, beta = [p_ref[k:k + 1, :] for k in range(5)]
    scale = lax.rsqrt(var + 1e-5) * gamma
    shift = (b - mean) * scale + beta
    dinv = dinv_ref[...]
    s = parts_ref[0] + parts_ref[1] + g_ref[...]
    h = jnp.maximum(s * dinv * scale + shift, 0.0)

    gid = lax.broadcasted_iota(jnp.int32, (1, G), 1)
    oh = (batch_ref[...] == gid).astype(jnp.float32)
    dn = (((0,), (0,)), ((), ()))
    blk_sums = lax.dot_general(oh, h, dn, precision=_HIGH)
    blk_cnts = lax.dot_general(oh, jnp.ones_like(h), dn, precision=_HIGH)

    @pl.when(i == 0)
    def _():
        sums[...] = jnp.zeros_like(sums)
        cnts[...] = jnp.zeros_like(cnts)

    sums[...] += blk_sums
    cnts[...] += blk_cnts

    @pl.when(i == NBLK - 1)
    def _():
        gemb = sums[...] / jnp.maximum(cnts[...], 1.0)
        o = jnp.maximum(
            jnp.dot(gemb, w1_ref[...], precision=_HIGH) + b1_ref[...], 0.0)
        out_ref[...] = jnp.dot(o, w2_ref[...], precision=_HIGH) + b2_ref[...]


def _tc_post(parts, g, dinv, p, batch2, w1, b1, w2, b2, n_cls):
    return pl.pallas_call(
        _post_body,
        grid=(NBLK,),
        in_specs=[
            pl.BlockSpec((NC, RB, H), lambda i: (0, i, 0)),
            pl.BlockSpec((RB, H), lambda i: (i, 0)),
            pl.BlockSpec((RB, 1), lambda i: (i, 0)),
            pl.BlockSpec((5, H), lambda i: (0, 0)),
            pl.BlockSpec((RB, 1), lambda i: (i, 0)),
            pl.BlockSpec((H, H // 2), lambda i: (0, 0)),
            pl.BlockSpec((1, H // 2), lambda i: (0, 0)),
            pl.BlockSpec((H // 2, n_cls), lambda i: (0, 0)),
            pl.BlockSpec((1, n_cls), lambda i: (0, 0)),
        ],
        out_specs=pl.BlockSpec((G, n_cls), lambda i: (0, 0)),
        out_shape=jax.ShapeDtypeStruct((G, n_cls), jnp.float32),
        scratch_shapes=[
            pltpu.VMEM((G, H), jnp.float32),
            pltpu.VMEM((G, H), jnp.float32),
        ],
        compiler_params=pltpu.CompilerParams(
            dimension_semantics=("arbitrary",)),
    )(parts, g, dinv, p, batch2, w1, b1, w2, b2)


def kernel(x, edge_index, batch, W_in, b_in, Ws, bs, gammas, betas,
           bn_means, bn_vars, W1, b1, W2, b2):
    n_cls = W2.shape[1]
    src = edge_index[0]
    dst = edge_index[1]

    npad_e = EPAD - E
    pad_i = jnp.arange(npad_e, dtype=jnp.int32)
    src_p = jnp.concatenate([src, pad_i % N])
    dst_p = jnp.concatenate([dst, N + (pad_i % (NPAD - N))])
    src3 = src_p.reshape(NW, NCHUNK, CH)
    dst3 = dst_p.reshape(NW, NCHUNK, CH)

    degp = _sc_degree(dst3).reshape(NC, NPAD)
    degp_t = degp.T

    g, dinv = _tc_pre(x, W_in, b_in.reshape(1, H), Ws[0], degp_t)

    ps = [jnp.stack([bs[i], bn_means[i], bn_vars[i], gammas[i], betas[i]])
          for i in range(L)]

    for i in range(L - 1):
        parts = _sc_aggregate(g, src3, dst3)
        g = _tc_mid(parts, g, dinv, ps[i], Ws[i + 1])

    parts = _sc_aggregate(g, src3, dst3)
    out = _tc_post(parts, g, dinv, ps[L - 1], batch.reshape(N, 1),
                   W1, b1.reshape(1, H // 2), W2, b2.reshape(1, n_cls), n_cls)
    return (out, jnp.float32(0.0))


# speedup vs baseline: 29.2389x; 1.0191x over previous
import functools

import jax
import jax.numpy as jnp
from jax import lax
from jax.experimental import pallas as pl
from jax.experimental.pallas import tpu as pltpu
from jax.experimental.pallas import tpu_sc as plsc

N = 10000
E = 320000
H = 128
G = 64
L = 4

NC = 2
NS = 16
NW = NC * NS
CH = 96
NCHUNK = 105
PH = 48
PHASES = ((0, 48), (48, 48), (96, 9))
EPAD = NW * NCHUNK * CH
NPAD = 10112
RPT = NPAD // NS
ZR = 32

_HIGH = jax.lax.Precision.DEFAULT


def _sc_mesh():
    return plsc.VectorSubcoreMesh(core_axis_name="c", subcore_axis_name="s")


def _deg_body(dst_hbm, out_hbm, dstall, ones_v, zv, deg_sh, sem, semz):
    cid = lax.axis_index("c")
    sid = lax.axis_index("s")
    wid = sid * NC + cid

    for k in range(RPT // 16 + 1):
        zv[pl.ds(k * 16, 16)] = jnp.zeros((16,), jnp.float32)
    for k in range(CH // 16):
        ones_v[pl.ds(k * 16, 16)] = jnp.ones((16,), jnp.float32)
    pltpu.async_copy(zv.at[pl.ds(0, RPT)],
                     deg_sh.at[pl.ds(sid * RPT, RPT)], semz).wait()
    plsc.subcore_barrier()

    pltpu.sync_copy(dst_hbm.at[wid], dstall)

    def body(c, _):
        pltpu.async_copy(ones_v, deg_sh.at[dstall.at[c]], sem, add=True)
        return 0
    lax.fori_loop(0, NCHUNK, body, 0)

    def drain(c, _):
        pltpu.make_async_copy(ones_v, deg_sh.at[dstall.at[0]], sem).wait()
        return 0
    lax.fori_loop(0, NCHUNK, drain, 0)
    plsc.subcore_barrier()

    pltpu.sync_copy(deg_sh.at[pl.ds(sid * RPT, RPT)], zv.at[pl.ds(0, RPT)])
    pltpu.sync_copy(zv.at[pl.ds(0, RPT)],
                    out_hbm.at[pl.ds(cid * NPAD + sid * RPT, RPT)])


def _sc_degree(dst3):
    kfn = pl.kernel(
        _deg_body,
        out_type=jax.ShapeDtypeStruct((NC * NPAD,), jnp.float32),
        mesh=_sc_mesh(),
        scratch_types=[
            pltpu.VMEM((NCHUNK, CH), jnp.int32),
            pltpu.VMEM((CH,), jnp.float32),
            pltpu.VMEM(((RPT // 16 + 1) * 16,), jnp.float32),
            pltpu.VMEM_SHARED((NPAD,), jnp.float32),
            pltpu.SemaphoreType.DMA,
            pltpu.SemaphoreType.DMA,
        ],
    )
    return kfn(dst3)


def _agg_body(g_hbm, src_hbm, dst_hbm, out_hbm,
              srcall, dstall, rows0, rows1, rows2, s_sh,
              sem_g0, sem_g1, sem_g2, sem_s0, sem_s1, sem_s2, semz):
    cid = lax.axis_index("c")
    sid = lax.axis_index("s")
    wid = sid * NC + cid

    for r in range(ZR):
        for k in range(8):
            rows2[r, pl.ds(k * 16, 16)] = jnp.zeros((16,), jnp.float32)

    def zbody(t, _):
        pltpu.async_copy(rows2.at[pl.ds(0, ZR)],
                         s_sh.at[pl.ds(sid * RPT + t * ZR, ZR)], semz)
        return 0
    nfull = RPT // ZR
    lax.fori_loop(0, nfull, zbody, 0)
    tail = RPT - nfull * ZR
    if tail:
        pltpu.async_copy(rows2.at[pl.ds(0, tail)],
                         s_sh.at[pl.ds(sid * RPT + nfull * ZR, tail)], semz)

    R = (rows0, rows1, rows2)
    SG = (sem_g0, sem_g1, sem_g2)
    SS = (sem_s0, sem_s1, sem_s2)

    def gather(c, p):
        pltpu.async_copy(g_hbm.at[srcall.at[c]], R[p], SG[p])

    def wait_g(p):
        pltpu.make_async_copy(g_hbm.at[srcall.at[0]], R[p], SG[p]).wait()

    def scat(c, p):
        pltpu.async_copy(R[p], s_sh.at[dstall.at[c]], SS[p], add=True)

    def wait_s(p):
        pltpu.make_async_copy(R[p], s_sh.at[dstall.at[0]], SS[p]).wait()

    def run_phase(base, n, prologue_done=False):
        if not prologue_done:
            pltpu.sync_copy(src_hbm.at[wid, pl.ds(base, n)],
                            srcall.at[pl.ds(0, n)])
            pltpu.sync_copy(dst_hbm.at[wid, pl.ds(base, n)],
                            dstall.at[pl.ds(0, n)])
            gather(0, 0)
            gather(1, 1)

        def body(j, _):
            a = 3 * j
            wait_g(0)
            scat(a, 0)

            @pl.when(j > 0)
            def _():
                wait_s(2)
            gather(a + 2, 2)
            wait_g(1)
            scat(a + 1, 1)

            @pl.when(a + 3 < n)
            def _():
                wait_s(0)
                gather(a + 3, 0)
            wait_g(2)
            scat(a + 2, 2)

            @pl.when(a + 4 < n)
            def _():
                wait_s(1)
                gather(a + 4, 1)
            return 0

        lax.fori_loop(0, n // 3, body, 0)
        wait_s(0)
        wait_s(1)
        wait_s(2)

    base0, n0 = PHASES[0]
    pltpu.sync_copy(src_hbm.at[wid, pl.ds(base0, n0)],
                    srcall.at[pl.ds(0, n0)])
    pltpu.sync_copy(dst_hbm.at[wid, pl.ds(base0, n0)],
                    dstall.at[pl.ds(0, n0)])
    gather(0, 0)
    gather(1, 1)
    for _ in range(nfull):
        pltpu.make_async_copy(rows2.at[pl.ds(0, ZR)],
                              s_sh.at[pl.ds(0, ZR)], semz).wait()
    if tail:
        pltpu.make_async_copy(rows2.at[pl.ds(0, tail)],
                              s_sh.at[pl.ds(0, tail)], semz).wait()
    plsc.subcore_barrier()

    run_phase(base0, n0, prologue_done=True)
    for base, n in PHASES[1:]:
        run_phase(base, n)
    plsc.subcore_barrier()

    npc = (RPT + CH - 1) // CH
    for t in range(npc):
        rows = rows0 if t % 2 == 0 else rows1
        sem_r = sem_g0 if t % 2 == 0 else sem_g1
        sem_w = sem_s0 if t % 2 == 0 else sem_s1
        nrows = min(CH, RPT - t * CH)
        if t >= 2:
            pltpu.make_async_copy(
                rows.at[pl.ds(0, min(CH, RPT - (t - 2) * CH))],
                out_hbm.at[cid, pl.ds(0, min(CH, RPT - (t - 2) * CH))],
                sem_w).wait()
        pltpu.async_copy(s_sh.at[pl.ds(sid * RPT + t * CH, nrows)],
                         rows.at[pl.ds(0, nrows)], sem_r).wait()
        pltpu.async_copy(rows.at[pl.ds(0, nrows)],
                         out_hbm.at[cid, pl.ds(sid * RPT + t * CH, nrows)],
                         sem_w)
    for t in (npc - 2, npc - 1):
        rows = rows0 if t % 2 == 0 else rows1
        sem_w = sem_s0 if t % 2 == 0 else sem_s1
        nrows = min(CH, RPT - t * CH)
        pltpu.make_async_copy(rows.at[pl.ds(0, nrows)],
                              out_hbm.at[cid, pl.ds(0, nrows)], sem_w).wait()


def _sc_aggregate(g, src3, dst3):
    kfn = pl.kernel(
        _agg_body,
        out_type=jax.ShapeDtypeStruct((NC, NPAD, H), jnp.float32),
        mesh=_sc_mesh(),
        scratch_types=[
            pltpu.VMEM((PH, CH), jnp.int32),
            pltpu.VMEM((PH, CH), jnp.int32),
            pltpu.VMEM((CH, H), jnp.float32),
            pltpu.VMEM((CH, H), jnp.float32),
            pltpu.VMEM((CH, H), jnp.float32),
            pltpu.VMEM_SHARED((NPAD, H), jnp.float32),
            pltpu.SemaphoreType.DMA,
            pltpu.SemaphoreType.DMA,
            pltpu.SemaphoreType.DMA,
            pltpu.SemaphoreType.DMA,
            pltpu.SemaphoreType.DMA,
            pltpu.SemaphoreType.DMA,
            pltpu.SemaphoreType.DMA,
        ],
    )
    return kfn(g, src3, dst3)


RB = 2000
NBLK = N // RB


def _pre_body(x_ref, win_ref, bin_ref, w0_ref, degp_ref, g_ref, dinv_ref):
    deg = degp_ref[:, 0:1] + degp_ref[:, 1:2] + 1.0
    dinv = lax.rsqrt(jnp.maximum(deg, 1.0))
    h = jnp.maximum(
        jnp.dot(x_ref[...], win_ref[...], precision=_HIGH) + bin_ref[...], 0.0)
    g_ref[...] = jnp.dot(h, w0_ref[...], precision=_HIGH) * dinv
    dinv_ref[...] = dinv


def _tc_pre(x, w_in, b_in, w0, degp_t):
    return pl.pallas_call(
        _pre_body,
        grid=(NBLK,),
        in_specs=[
            pl.BlockSpec((RB, H), lambda i: (i, 0)),
            pl.BlockSpec((H, H), lambda i: (0, 0)),
            pl.BlockSpec((1, H), lambda i: (0, 0)),
            pl.BlockSpec((H, H), lambda i: (0, 0)),
            pl.BlockSpec((RB, NC), lambda i: (i, 0)),
        ],
        out_specs=[
            pl.BlockSpec((RB, H), lambda i: (i, 0)),
            pl.BlockSpec((RB, 1), lambda i: (i, 0)),
        ],
        out_shape=[
            jax.ShapeDtypeStruct((N, H), jnp.float32),
            jax.ShapeDtypeStruct((N, 1), jnp.float32),
        ],
    )(x, w_in, b_in, w0, degp_t)


def _mid_body(parts_ref, g_ref, dinv_ref, p_ref, w_ref, out_ref):
    b, mean, var, gamma, beta = [p_ref[k:k + 1, :] for k in range(5)]
    scale = lax.rsqrt(var + 1e-5) * gamma
    shift = (b - mean) * scale + beta
    dinv = dinv_ref[...]
    s = parts_ref[0] + parts_ref[1] + g_ref[...]
    h = jnp.maximum(s * dinv * scale + shift, 0.0)
    out_ref[...] = jnp.dot(h, w_ref[...], precision=_HIGH) * dinv


def _tc_mid(parts, g, dinv, p, w_next):
    return pl.pallas_call(
        _mid_body,
        grid=(NBLK,),
        in_specs=[
            pl.BlockSpec((NC, RB, H), lambda i: (0, i, 0)),
            pl.BlockSpec((RB, H), lambda i: (i, 0)),
            pl.BlockSpec((RB, 1), lambda i: (i, 0)),
            pl.BlockSpec((5, H), lambda i: (0, 0)),
            pl.BlockSpec((H, H), lambda i: (0, 0)),
        ],
        out_specs=pl.BlockSpec((RB, H), lambda i: (i, 0)),
        out_shape=jax.ShapeDtypeStruct((N, H), jnp.float32),
    )(parts, g, dinv, p, w_next)


def _post_body(parts_ref, g_ref, dinv_ref, p_ref, batch_ref,
               w1_ref, b1_ref, w2_ref, b2_ref, out_ref, sums, cnts):
    i = pl.program_id(0)
    b, mean, var, gamma, beta = [p_ref[k:k + 1, :] for k in range(5)]
    scale = lax.rsqrt(var + 1e-5) * gamma
    shift = (b - mean) * scale + beta
    dinv = dinv_ref[...]
    s = parts_ref[0] + parts_ref[1] + g_ref[...]
    h = jnp.maximum(s * dinv * scale + shift, 0.0)

    gid = lax.broadcasted_iota(jnp.int32, (1, G), 1)
    oh = (batch_ref[...] == gid).astype(jnp.float32)
    dn = (((0,), (0,)), ((), ()))
    blk_sums = lax.dot_general(oh, h, dn, precision=_HIGH)
    blk_cnts = lax.dot_general(oh, jnp.ones_like(h), dn, precision=_HIGH)

    @pl.when(i == 0)
    def _():
        sums[...] = jnp.zeros_like(sums)
        cnts[...] = jnp.zeros_like(cnts)

    sums[...] += blk_sums
    cnts[...] += blk_cnts

    @pl.when(i == NBLK - 1)
    def _():
        gemb = sums[...] / jnp.maximum(cnts[...], 1.0)
        o = jnp.maximum(
            jnp.dot(gemb, w1_ref[...], precision=_HIGH) + b1_ref[...], 0.0)
        out_ref[...] = jnp.dot(o, w2_ref[...], precision=_HIGH) + b2_ref[...]


def _tc_post(parts, g, dinv, p, batch2, w1, b1, w2, b2, n_cls):
    return pl.pallas_call(
        _post_body,
        grid=(NBLK,),
        in_specs=[
            pl.BlockSpec((NC, RB, H), lambda i: (0, i, 0)),
            pl.BlockSpec((RB, H), lambda i: (i, 0)),
            pl.BlockSpec((RB, 1), lambda i: (i, 0)),
            pl.BlockSpec((5, H), lambda i: (0, 0)),
            pl.BlockSpec((RB, 1), lambda i: (i, 0)),
            pl.BlockSpec((H, H // 2), lambda i: (0, 0)),
            pl.BlockSpec((1, H // 2), lambda i: (0, 0)),
            pl.BlockSpec((H // 2, n_cls), lambda i: (0, 0)),
            pl.BlockSpec((1, n_cls), lambda i: (0, 0)),
        ],
        out_specs=pl.BlockSpec((G, n_cls), lambda i: (0, 0)),
        out_shape=jax.ShapeDtypeStruct((G, n_cls), jnp.float32),
        scratch_shapes=[
            pltpu.VMEM((G, H), jnp.float32),
            pltpu.VMEM((G, H), jnp.float32),
        ],
        compiler_params=pltpu.CompilerParams(
            dimension_semantics=("arbitrary",)),
    )(parts, g, dinv, p, batch2, w1, b1, w2, b2)


def kernel(x, edge_index, batch, W_in, b_in, Ws, bs, gammas, betas,
           bn_means, bn_vars, W1, b1, W2, b2):
    n_cls = W2.shape[1]
    src = edge_index[0]
    dst = edge_index[1]

    npad_e = EPAD - E
    pad_i = jnp.arange(npad_e, dtype=jnp.int32)
    src_p = jnp.concatenate([src, pad_i % N])
    dst_p = jnp.concatenate([dst, N + (pad_i % (NPAD - N))])
    src3 = src_p.reshape(NW, NCHUNK, CH)
    dst3 = dst_p.reshape(NW, NCHUNK, CH)

    degp = _sc_degree(dst3).reshape(NC, NPAD)
    degp_t = degp.T

    g, dinv = _tc_pre(x, W_in, b_in.reshape(1, H), Ws[0], degp_t)

    ps = [jnp.stack([bs[i], bn_means[i], bn_vars[i], gammas[i], betas[i]])
          for i in range(L)]

    for i in range(L - 1):
        parts = _sc_aggregate(g, src3, dst3)
        g = _tc_mid(parts, g, dinv, ps[i], Ws[i + 1])

    parts = _sc_aggregate(g, src3, dst3)
    out = _tc_post(parts, g, dinv, ps[L - 1], batch.reshape(N, 1),
                   W1, b1.reshape(1, H // 2), W2, b2.reshape(1, n_cls), n_cls)
    return (out, jnp.float32(0.0))
